# Initial kernel scaffold; baseline (speedup 1.0000x reference)
import jax
import jax.numpy as jnp
from jax.experimental import pallas as pl


def kernel(x, EdgeID, EdgeAttr, batch, emb_W, emb_b, lin1_W, lin1_b, lin2_W, lin3_W, lin3_b, bn_gamma, bn_beta, fc1_W, fc1_b, fc2_W, fc2_b):
    raise NotImplementedError("write your pallas kernel here")


# speedup vs baseline: 3.5296x; 3.5296x over previous
import functools

import jax
import jax.numpy as jnp
from jax import lax
from jax.experimental import pallas as pl
from jax.experimental.pallas import tpu as pltpu
from jax.experimental.pallas import tpu_sc as plsc

N = 50000
E = 800000
G = 512
D = 64
H = 32
NC = 2
NS = 16
C = 128
RB = 1000
NB = N // RB

_mesh = plsc.VectorSubcoreMesh(core_axis_name="c", subcore_axis_name="s",
                               num_cores=NC, num_subcores=NS)


def _zero_shared(zrow_hbm, acc, s, rows_per_chunk, n_rows):
    n_chunks = n_rows // rows_per_chunk

    def zcopy(k, _):
        idx = s + k * NS

        @pl.when(idx < n_chunks)
        def _():
            pltpu.sync_copy(zrow_hbm, acc.at[pl.ds(idx * rows_per_chunk,
                                                   rows_per_chunk)])
        return 0
    lax.fori_loop(0, pl.cdiv(n_chunks, NS), zcopy, 0)


@functools.partial(
    pl.kernel,
    out_type=jax.ShapeDtypeStruct((NC, N, 16), jnp.float32),
    mesh=_mesh,
    compiler_params=pltpu.CompilerParams(use_tc_tiling_on_sc=False),
    scratch_types=[
        pltpu.VMEM((C,), jnp.int32),
        pltpu.VMEM((C,), jnp.float32),
        pltpu.VMEM((C, 16), jnp.float32),
        pltpu.VMEM_SHARED((N, 16), jnp.float32),
        pltpu.SemaphoreType.DMA,
    ],
)
def _deg_kernel(dst_hbm, w_hbm, z16_hbm, out_hbm, dstb, wb, msgb, acc, sem):
    c = lax.axis_index("c")
    s = lax.axis_index("s")
    _zero_shared(z16_hbm, acc, s, RB, N)
    plsc.subcore_barrier()

    e_half = E // NC
    n_chunks = e_half // C

    def chunk_body(k, _):
        chunk = s + k * NS

        @pl.when(chunk < n_chunks)
        def _():
            base = c * e_half + chunk * C
            pltpu.sync_copy(dst_hbm.at[pl.ds(base, C)], dstb)
            pltpu.sync_copy(w_hbm.at[pl.ds(base, C)], wb)

            def scale(g, _):
                w16 = wb[pl.ds(g * 16, 16)]
                for l in range(16):
                    msgb[g * 16 + l, 0:16] = jnp.full((16,), w16[l],
                                                      jnp.float32)
                return 0
            lax.fori_loop(0, C // 16, scale, 0)
            pltpu.sync_copy(msgb, acc.at[dstb], add=True)
        return 0
    lax.fori_loop(0, pl.cdiv(n_chunks, NS), chunk_body, 0)
    plsc.subcore_barrier()

    def out_copy(k, _):
        idx = s + k * NS

        @pl.when(idx < N // RB)
        def _():
            pltpu.sync_copy(acc.at[pl.ds(idx * RB, RB)],
                            out_hbm.at[c, pl.ds(idx * RB, RB)])
        return 0
    lax.fori_loop(0, pl.cdiv(N // RB, NS), out_copy, 0)


@functools.partial(
    pl.kernel,
    out_type=jax.ShapeDtypeStruct((NC * N, H), jnp.float32),
    mesh=_mesh,
    compiler_params=pltpu.CompilerParams(use_tc_tiling_on_sc=False),
    scratch_types=[
        pltpu.VMEM((C,), jnp.int32),
        pltpu.VMEM((C,), jnp.int32),
        pltpu.VMEM((C,), jnp.int32),
        pltpu.VMEM((C,), jnp.float32),
        pltpu.VMEM((C, H), jnp.float32),
        pltpu.VMEM_SHARED((N, H), jnp.float32),
        pltpu.SemaphoreType.DMA,
    ],
)
def _spmm_kernel(a2_hbm, src_hbm, dst_hbm, w_hbm, z32_hbm, out_hbm,
                 srcb, srcadj, dstb, wb, rows, acc, sem):
    c = lax.axis_index("c")
    s = lax.axis_index("s")
    _zero_shared(z32_hbm, acc, s, RB, N)
    plsc.subcore_barrier()

    n_chunks = E // C
    row_off = c * N

    def chunk_body(k, _):
        chunk = s + k * NS

        @pl.when(chunk < n_chunks)
        def _():
            base = chunk * C
            pltpu.sync_copy(src_hbm.at[pl.ds(base, C)], srcb)
            pltpu.sync_copy(dst_hbm.at[pl.ds(base, C)], dstb)
            pltpu.sync_copy(w_hbm.at[pl.ds(base, C)], wb)

            def adj(j, _):
                srcadj[pl.ds(j * 16, 16)] = srcb[pl.ds(j * 16, 16)] + row_off
                return 0
            lax.fori_loop(0, C // 16, adj, 0)
            pltpu.async_copy(a2_hbm.at[srcadj], rows, sem).wait()

            def scale(g, _):
                w16 = wb[pl.ds(g * 16, 16)]
                for l in range(16):
                    e = g * 16 + l
                    rows[e, 0:16] = rows[e, 0:16] * w16[l]
                    rows[e, 16:32] = rows[e, 16:32] * w16[l]
                return 0
            lax.fori_loop(0, C // 16, scale, 0)
            pltpu.sync_copy(rows, acc.at[dstb], add=True)
        return 0
    lax.fori_loop(0, pl.cdiv(n_chunks, NS), chunk_body, 0)
    plsc.subcore_barrier()

    def out_copy(k, _):
        idx = s + k * NS

        @pl.when(idx < N // RB)
        def _():
            pltpu.sync_copy(acc.at[pl.ds(idx * RB, RB)],
                            out_hbm.at[pl.ds(row_off + idx * RB, RB)])
        return 0
    lax.fori_loop(0, pl.cdiv(N // RB, NS), out_copy, 0)


_NFULL = N // C
_REM = N - _NFULL * C


@functools.partial(
    pl.kernel,
    out_type=(jax.ShapeDtypeStruct((NC, G, H), jnp.float32),
              jax.ShapeDtypeStruct((NC, G, 16), jnp.float32)),
    mesh=_mesh,
    compiler_params=pltpu.CompilerParams(use_tc_tiling_on_sc=False),
    scratch_types=[
        pltpu.VMEM((C,), jnp.int32),
        pltpu.VMEM((C, H), jnp.float32),
        pltpu.VMEM((C, 16), jnp.float32),
        pltpu.VMEM((_REM,), jnp.int32),
        pltpu.VMEM((_REM, H), jnp.float32),
        pltpu.VMEM((_REM, 16), jnp.float32),
        pltpu.VMEM_SHARED((G, H), jnp.float32),
        pltpu.VMEM_SHARED((G, 16), jnp.float32),
        pltpu.SemaphoreType.DMA,
    ],
)
def _pool_kernel(h2_hbm, batch_hbm, z32_hbm, z16_hbm, ones_hbm,
                 sum_hbm, cnt_hbm,
                 bb, rows, ones, bb2, rows2, ones2, accS, accC, sem):
    c = lax.axis_index("c")
    s = lax.axis_index("s")

    @pl.when(s == 0)
    def _():
        pltpu.sync_copy(z32_hbm.at[pl.ds(0, G)], accS)
        pltpu.sync_copy(z16_hbm.at[pl.ds(0, G)], accC)

    pltpu.sync_copy(ones_hbm, ones)
    pltpu.sync_copy(ones_hbm.at[pl.ds(0, _REM)], ones2)
    plsc.subcore_barrier()

    row_off = c * N

    def chunk_body(k, _):
        chunk = s + k * NS

        @pl.when(chunk < _NFULL)
        def _():
            base = chunk * C
            pltpu.sync_copy(batch_hbm.at[pl.ds(base, C)], bb)
            pltpu.sync_copy(h2_hbm.at[pl.ds(row_off + base, C)], rows)
            pltpu.sync_copy(rows, accS.at[bb], add=True)
            pltpu.sync_copy(ones, accC.at[bb], add=True)
        return 0
    lax.fori_loop(0, pl.cdiv(_NFULL, NS), chunk_body, 0)

    @pl.when(s == NS - 1)
    def _():
        base = _NFULL * C
        pltpu.sync_copy(batch_hbm.at[pl.ds(base, _REM)], bb2)
        pltpu.sync_copy(h2_hbm.at[pl.ds(row_off + base, _REM)], rows2)
        pltpu.sync_copy(rows2, accS.at[bb2], add=True)
        pltpu.sync_copy(ones2, accC.at[bb2], add=True)
    plsc.subcore_barrier()

    @pl.when(s == 0)
    def _():
        pltpu.sync_copy(accS, sum_hbm.at[c])
        pltpu.sync_copy(accC, cnt_hbm.at[c])


def _mm_first_body(x_ref, embW_ref, embb_ref, W1_ref, b1_ref, W2_ref, W3_ref,
                   b3_ref, deg_ref, a_ref, rest_ref):
    h = jnp.dot(x_ref[...], embW_ref[...],
                preferred_element_type=jnp.float32) + embb_ref[...]
    deg = (deg_ref[0, :, 0:1] + deg_ref[1, :, 0:1])
    a_ref[...] = jnp.dot(h, W1_ref[0], preferred_element_type=jnp.float32) \
        + b1_ref[0]
    rest_ref[...] = (jnp.dot(h, W3_ref[0], preferred_element_type=jnp.float32)
                     + b3_ref[0]
                     - deg * jnp.dot(h, W2_ref[0],
                                     preferred_element_type=jnp.float32))


def _mm_body(hA_ref, hB_ref, ssum_ref, ssq_ref, gam_ref, bet_ref,
             W1_ref, b1_ref, W2_ref, W3_ref, b3_ref, deg_ref,
             a_ref, rest_ref):
    inv_n = 1.0 / N
    meanA = ssum_ref[0] * inv_n
    meanB = ssum_ref[1] * inv_n
    varA = ssq_ref[0] * inv_n - meanA * meanA
    varB = ssq_ref[1] * inv_n - meanB * meanB
    scaleA = gam_ref[0] * lax.rsqrt(varA + 1e-5)
    scaleB = gam_ref[1] * lax.rsqrt(varB + 1e-5)
    shiftA = bet_ref[0] - meanA * scaleA
    shiftB = bet_ref[1] - meanB * scaleB
    hA = jnp.maximum(hA_ref[...] * scaleA + shiftA, 0.0)
    hB = jnp.maximum(hB_ref[...] * scaleB + shiftB, 0.0)
    h = jnp.concatenate([hA, hB], axis=1)
    deg = (deg_ref[0, :, 0:1] + deg_ref[1, :, 0:1])
    a_ref[...] = jnp.dot(h, W1_ref[0], preferred_element_type=jnp.float32) \
        + b1_ref[0]
    rest_ref[...] = (jnp.dot(h, W3_ref[0], preferred_element_type=jnp.float32)
                     + b3_ref[0]
                     - deg * jnp.dot(h, W2_ref[0],
                                     preferred_element_type=jnp.float32))


def _combine_body(agg_ref, rest_ref, h_ref, ssum_ref, ssq_ref):
    i = pl.program_id(1)
    hp = agg_ref[...] + rest_ref[...]
    h_ref[...] = hp
    bs = jnp.sum(hp, axis=0, keepdims=True)
    bq = jnp.sum(hp * hp, axis=0, keepdims=True)

    @pl.when(i == 0)
    def _():
        ssum_ref[...] = bs[None]
        ssq_ref[...] = bq[None]

    @pl.when(i > 0)
    def _():
        ssum_ref[...] += bs[None]
        ssq_ref[...] += bq[None]


def _norm_body(h_ref, ssum_ref, ssq_ref, gam_ref, bet_ref, out_ref):
    inv_n = 1.0 / N
    mean = ssum_ref[0] * inv_n
    var = ssq_ref[0] * inv_n - mean * mean
    scale = gam_ref[0] * lax.rsqrt(var + 1e-5)
    shift = bet_ref[0] - mean * scale
    out_ref[...] = jnp.maximum(h_ref[...] * scale + shift, 0.0)


def _head_body(sum_ref, cnt_ref, W1_ref, b1_ref, W2_ref, b2_ref, out_ref):
    sums = jnp.concatenate([sum_ref[0], sum_ref[1]], axis=1)
    cnt = cnt_ref[0, :, 0:1]
    gx = sums / jnp.maximum(cnt, 1.0)
    p = jnp.maximum(jnp.dot(gx, W1_ref[...],
                            preferred_element_type=jnp.float32)
                    + b1_ref[...], 0.0)
    out_ref[...] = jnp.dot(p, W2_ref[...],
                           preferred_element_type=jnp.float32) + b2_ref[...]


def _row_spec(im):
    return pl.BlockSpec((RB, H), im)


_W_spec = pl.BlockSpec((1, D, H), lambda c, i: (c, 0, 0))
_b_spec = pl.BlockSpec((1, 1, H), lambda c, i: (c, 0, 0))
_deg_spec = pl.BlockSpec((NC, RB, 16), lambda c, i: (0, i, 0))
_stat_spec = pl.BlockSpec((1, 1, H), lambda c, i: (c, 0, 0))
_f32 = jnp.float32


def kernel(x, EdgeID, EdgeAttr, batch, emb_W, emb_b, lin1_W, lin1_b, lin2_W,
           lin3_W, lin3_b, bn_gamma, bn_beta, fc1_W, fc1_b, fc2_W, fc2_b):
    src = EdgeID[0].astype(jnp.int32)
    dst = EdgeID[1].astype(jnp.int32)
    batch = batch.astype(jnp.int32)

    z32 = jnp.zeros((RB, H), jnp.float32)
    z16 = jnp.zeros((RB, 16), jnp.float32)
    ones_rows = jnp.ones((C, 16), jnp.float32)

    deg2 = _deg_kernel(dst, EdgeAttr, z16)

    def stack_w(W):
        return jnp.stack([W[:, :H], W[:, H:]], axis=0)

    def stack_b(b):
        return b.reshape(NC, 1, H)

    out_ab = [jax.ShapeDtypeStruct((NC * N, H), _f32)] * 2
    ab_specs = [_row_spec(lambda c, i: (c * NB + i, 0))] * 2

    a2, rest2 = pl.pallas_call(
        _mm_first_body,
        grid=(NC, NB),
        in_specs=[
            pl.BlockSpec((RB, 4), lambda c, i: (i, 0)),
            pl.BlockSpec((4, D), lambda c, i: (0, 0)),
            pl.BlockSpec((1, D), lambda c, i: (0, 0)),
            _W_spec, _b_spec, _W_spec, _W_spec, _b_spec, _deg_spec,
        ],
        out_specs=ab_specs,
        out_shape=out_ab,
    )(x, emb_W, emb_b.reshape(1, D), stack_w(lin1_W[0]), stack_b(lin1_b[0]),
      stack_w(lin2_W[0]), stack_w(lin3_W[0]), stack_b(lin3_b[0]), deg2)

    hpre2 = ssum = ssq = None
    for i in range(3):
        agg2 = _spmm_kernel(a2, src, dst, EdgeAttr, z32)
        hpre2, ssum, ssq = pl.pallas_call(
            _combine_body,
            grid=(NC, NB),
            in_specs=ab_specs,
            out_specs=[ab_specs[0], _stat_spec, _stat_spec],
            out_shape=[jax.ShapeDtypeStruct((NC * N, H), _f32),
                       jax.ShapeDtypeStruct((NC, 1, H), _f32),
                       jax.ShapeDtypeStruct((NC, 1, H), _f32)],
        )(agg2, rest2)
        if i < 2:
            a2, rest2 = pl.pallas_call(
                _mm_body,
                grid=(NC, NB),
                in_specs=[
                    _row_spec(lambda c, i: (i, 0)),
                    _row_spec(lambda c, i: (NB + i, 0)),
                    pl.BlockSpec((NC, 1, H), lambda c, i: (0, 0, 0)),
                    pl.BlockSpec((NC, 1, H), lambda c, i: (0, 0, 0)),
                    pl.BlockSpec((NC, 1, H), lambda c, i: (0, 0, 0)),
                    pl.BlockSpec((NC, 1, H), lambda c, i: (0, 0, 0)),
                    _W_spec, _b_spec, _W_spec, _W_spec, _b_spec, _deg_spec,
                ],
                out_specs=ab_specs,
                out_shape=out_ab,
            )(hpre2, hpre2, ssum, ssq,
              bn_gamma[i].reshape(NC, 1, H), bn_beta[i].reshape(NC, 1, H),
              stack_w(lin1_W[i + 1]), stack_b(lin1_b[i + 1]),
              stack_w(lin2_W[i + 1]), stack_w(lin3_W[i + 1]),
              stack_b(lin3_b[i + 1]), deg2)

    hfin2 = pl.pallas_call(
        _norm_body,
        grid=(NC, NB),
        in_specs=[ab_specs[0], _stat_spec, _stat_spec, _stat_spec, _stat_spec],
        out_specs=ab_specs[0],
        out_shape=jax.ShapeDtypeStruct((NC * N, H), _f32),
    )(hpre2, ssum, ssq, bn_gamma[2].reshape(NC, 1, H),
      bn_beta[2].reshape(NC, 1, H))

    gsum, gcnt = _pool_kernel(hfin2, batch, z32, z16, ones_rows)

    out = pl.pallas_call(
        _head_body,
        in_specs=[
            pl.BlockSpec((NC, G, H), lambda: (0, 0, 0)),
            pl.BlockSpec((NC, G, 16), lambda: (0, 0, 0)),
            pl.BlockSpec((D, D), lambda: (0, 0)),
            pl.BlockSpec((1, D), lambda: (0, 0)),
            pl.BlockSpec((D, 3), lambda: (0, 0)),
            pl.BlockSpec((1, 3), lambda: (0, 0)),
        ],
        out_specs=pl.BlockSpec((G, 3), lambda: (0, 0)),
        out_shape=jax.ShapeDtypeStruct((G, 3), _f32),
    )(gsum, gcnt, fc1_W, fc1_b.reshape(1, D), fc2_W, fc2_b.reshape(1, 3))
    return out


# speedup vs baseline: 6.7563x; 1.9142x over previous
import functools

import jax
import jax.numpy as jnp
from jax import lax
from jax.experimental import pallas as pl
from jax.experimental.pallas import tpu as pltpu
from jax.experimental.pallas import tpu_sc as plsc

N = 50000
E = 800000
G = 512
D = 64
H = 32
NC = 2
NS = 16
C = 128
RB = 1000
NB = N // RB

_mesh = plsc.VectorSubcoreMesh(core_axis_name="c", subcore_axis_name="s",
                               num_cores=NC, num_subcores=NS)


def _zero_shared(zrow_hbm, acc, s, rows_per_chunk, n_rows):
    n_chunks = n_rows // rows_per_chunk

    def zcopy(k, _):
        idx = s + k * NS

        @pl.when(idx < n_chunks)
        def _():
            pltpu.sync_copy(zrow_hbm, acc.at[pl.ds(idx * rows_per_chunk,
                                                   rows_per_chunk)])
        return 0
    lax.fori_loop(0, pl.cdiv(n_chunks, NS), zcopy, 0)


@functools.partial(
    pl.kernel,
    out_type=jax.ShapeDtypeStruct((NC, N, 16), jnp.float32),
    mesh=_mesh,
    compiler_params=pltpu.CompilerParams(use_tc_tiling_on_sc=False, needs_layout_passes=False),
    scratch_types=[
        pltpu.VMEM((3 * 128,), jnp.int32),
        pltpu.VMEM((128,), jnp.int32),
        pltpu.VMEM((128, 16), jnp.float32),
        pltpu.VMEM_SHARED((N, 16), jnp.float32),
        pltpu.SemaphoreType.DMA,
    ],
)
def _deg_kernel(ebuf_hbm, z16_hbm, out_hbm, ebuf, dstb, msgb, acc, sem):
    c = lax.axis_index("c")
    s = lax.axis_index("s")
    _zero_shared(z16_hbm, acc, s, RB, N)
    plsc.subcore_barrier()

    kps = 6272 // NC // NS
    ch0 = c * (6272 // NC) + s * kps

    def chunk_body(k, _):
        pltpu.sync_copy(ebuf_hbm.at[ch0 + k], ebuf)

        def adj(j, _):
            dstb[pl.ds(j * 16, 16)] = ebuf[pl.ds(C + j * 16, 16)]
            return 0
        lax.fori_loop(0, C // 16, adj, 0)

        def scale(g, _):
            w16 = plsc.bitcast(ebuf[pl.ds(2 * C + g * 16, 16)], jnp.float32)
            for l in range(16):
                msgb[g * 16 + l, 0:16] = jnp.full((16,), w16[l], jnp.float32)
            return 0
        lax.fori_loop(0, C // 16, scale, 0)
        pltpu.sync_copy(msgb, acc.at[dstb], add=True)
        return 0
    lax.fori_loop(0, kps, chunk_body, 0)
    plsc.subcore_barrier()

    def out_copy(k, _):
        idx = s + k * NS

        @pl.when(idx < N // RB)
        def _():
            pltpu.sync_copy(acc.at[pl.ds(idx * RB, RB)],
                            out_hbm.at[c, pl.ds(idx * RB, RB)])
        return 0
    lax.fori_loop(0, pl.cdiv(N // RB, NS), out_copy, 0)


EP = 6272 * C
_NCHUNKS = EP // C
_KPS = _NCHUNKS // NS


@functools.partial(
    pl.kernel,
    out_type=jax.ShapeDtypeStruct((NC * N, H), jnp.float32),
    mesh=_mesh,
    compiler_params=pltpu.CompilerParams(use_tc_tiling_on_sc=False, needs_layout_passes=False),
    scratch_types=[
        [pltpu.VMEM((3 * C,), jnp.int32)] * 2,
        [pltpu.VMEM((C,), jnp.int32)] * 2,
        [pltpu.VMEM((C,), jnp.int32)] * 2,
        [pltpu.VMEM((C, H), jnp.float32)] * 2,
        [pltpu.SemaphoreType.DMA] * 2,
        [pltpu.SemaphoreType.DMA] * 2,
        [pltpu.SemaphoreType.DMA] * 2,
        pltpu.VMEM_SHARED((N, H), jnp.float32),
    ],
)
def _spmm_kernel(a2_hbm, ebuf_hbm, z32_hbm, out_hbm,
                 ebuf, srcadj, dstb, rows, sem_e, sem_g, sem_s, acc):
    c = lax.axis_index("c")
    s = lax.axis_index("s")
    _zero_shared(z32_hbm, acc, s, RB, N)
    plsc.subcore_barrier()

    row_off = c * N
    ch0 = s * _KPS

    def unpack(b, ebv):
        def adj(j, _):
            srcadj[b][pl.ds(j * 16, 16)] = ebv[pl.ds(j * 16, 16)] + row_off
            dstb[b][pl.ds(j * 16, 16)] = ebv[pl.ds(C + j * 16, 16)]
            return 0
        lax.fori_loop(0, C // 16, adj, 0)

    def scale(b, ebv):
        def body(g, _):
            w16 = plsc.bitcast(ebv[pl.ds(2 * C + g * 16, 16)], jnp.float32)
            for l in range(16):
                e = g * 16 + l
                rows[b][e, 0:16] = rows[b][e, 0:16] * w16[l]
                rows[b][e, 16:32] = rows[b][e, 16:32] * w16[l]
            return 0
        lax.fori_loop(0, C // 16, body, 0)

    pltpu.sync_copy(ebuf_hbm.at[ch0], ebuf[0])
    unpack(0, ebuf[0])
    pltpu.async_copy(a2_hbm.at[srcadj[0]], rows[0], sem_g[0])
    pltpu.async_copy(ebuf_hbm.at[ch0 + 1], ebuf[1], sem_e[1])

    def visit(k, b):
        nxt = 1 - b

        @pl.when(k + 1 < _KPS)
        def _():
            pltpu.make_async_copy(ebuf_hbm.at[ch0], ebuf[nxt],
                                  sem_e[nxt]).wait()

            @pl.when(k >= 1)
            def _():
                pltpu.make_async_copy(rows[nxt], acc.at[dstb[nxt]],
                                      sem_s[nxt]).wait()
            unpack(nxt, ebuf[nxt])
            pltpu.async_copy(a2_hbm.at[srcadj[nxt]], rows[nxt], sem_g[nxt])
        pltpu.make_async_copy(a2_hbm.at[srcadj[b]], rows[b], sem_g[b]).wait()
        scale(b, ebuf[b])

        @pl.when(k + 2 < _KPS)
        def _():
            pltpu.async_copy(ebuf_hbm.at[ch0 + k + 2], ebuf[b], sem_e[b])
        pltpu.async_copy(rows[b], acc.at[dstb[b]], sem_s[b], add=True)

    def pair(kk, _):
        visit(kk * 2, 0)
        visit(kk * 2 + 1, 1)
        return 0
    lax.fori_loop(0, _KPS // 2, pair, 0)
    pltpu.make_async_copy(rows[0], acc.at[dstb[0]], sem_s[0]).wait()
    pltpu.make_async_copy(rows[1], acc.at[dstb[1]], sem_s[1]).wait()
    plsc.subcore_barrier()

    def out_copy(k, _):
        idx = s + k * NS

        @pl.when(idx < N // RB)
        def _():
            pltpu.sync_copy(acc.at[pl.ds(idx * RB, RB)],
                            out_hbm.at[pl.ds(row_off + idx * RB, RB)])
        return 0
    lax.fori_loop(0, pl.cdiv(N // RB, NS), out_copy, 0)


_NFULL = N // C
_REM = N - _NFULL * C


@functools.partial(
    pl.kernel,
    out_type=(jax.ShapeDtypeStruct((NC, G, H), jnp.float32),
              jax.ShapeDtypeStruct((NC, G, 16), jnp.float32)),
    mesh=_mesh,
    compiler_params=pltpu.CompilerParams(use_tc_tiling_on_sc=False, needs_layout_passes=False),
    scratch_types=[
        pltpu.VMEM((C,), jnp.int32),
        pltpu.VMEM((C, H), jnp.float32),
        pltpu.VMEM((C, 16), jnp.float32),
        pltpu.VMEM((_REM,), jnp.int32),
        pltpu.VMEM((_REM, H), jnp.float32),
        pltpu.VMEM((_REM, 16), jnp.float32),
        pltpu.VMEM_SHARED((G, H), jnp.float32),
        pltpu.VMEM_SHARED((G, 16), jnp.float32),
        pltpu.SemaphoreType.DMA,
    ],
)
def _pool_kernel(h2_hbm, batch_hbm, z32_hbm, z16_hbm, ones_hbm,
                 sum_hbm, cnt_hbm,
                 bb, rows, ones, bb2, rows2, ones2, accS, accC, sem):
    c = lax.axis_index("c")
    s = lax.axis_index("s")

    @pl.when(s == 0)
    def _():
        pltpu.sync_copy(z32_hbm.at[pl.ds(0, G)], accS)
        pltpu.sync_copy(z16_hbm.at[pl.ds(0, G)], accC)

    pltpu.sync_copy(ones_hbm, ones)
    pltpu.sync_copy(ones_hbm.at[pl.ds(0, _REM)], ones2)
    plsc.subcore_barrier()

    row_off = c * N

    def chunk_body(k, _):
        chunk = s + k * NS

        @pl.when(chunk < _NFULL)
        def _():
            base = chunk * C
            pltpu.sync_copy(batch_hbm.at[pl.ds(base, C)], bb)
            pltpu.sync_copy(h2_hbm.at[pl.ds(row_off + base, C)], rows)
            pltpu.sync_copy(rows, accS.at[bb], add=True)
            pltpu.sync_copy(ones, accC.at[bb], add=True)
        return 0
    lax.fori_loop(0, pl.cdiv(_NFULL, NS), chunk_body, 0)

    @pl.when(s == NS - 1)
    def _():
        base = _NFULL * C
        pltpu.sync_copy(batch_hbm.at[pl.ds(base, _REM)], bb2)
        pltpu.sync_copy(h2_hbm.at[pl.ds(row_off + base, _REM)], rows2)
        pltpu.sync_copy(rows2, accS.at[bb2], add=True)
        pltpu.sync_copy(ones2, accC.at[bb2], add=True)
    plsc.subcore_barrier()

    @pl.when(s == 0)
    def _():
        pltpu.sync_copy(accS, sum_hbm.at[c])
        pltpu.sync_copy(accC, cnt_hbm.at[c])


def _mm_first_body(x_ref, embW_ref, embb_ref, W1_ref, b1_ref, W2_ref, W3_ref,
                   b3_ref, deg_ref, a_ref, rest_ref):
    h = jnp.dot(x_ref[...], embW_ref[...],
                preferred_element_type=jnp.float32) + embb_ref[...]
    deg = (deg_ref[0, :, 0:1] + deg_ref[1, :, 0:1])
    a_ref[...] = jnp.dot(h, W1_ref[0], preferred_element_type=jnp.float32) \
        + b1_ref[0]
    rest_ref[...] = (jnp.dot(h, W3_ref[0], preferred_element_type=jnp.float32)
                     + b3_ref[0]
                     - deg * jnp.dot(h, W2_ref[0],
                                     preferred_element_type=jnp.float32))


def _mm_body(hA_ref, hB_ref, ssum_ref, ssq_ref, gam_ref, bet_ref,
             W1_ref, b1_ref, W2_ref, W3_ref, b3_ref, deg_ref,
             a_ref, rest_ref):
    inv_n = 1.0 / N
    meanA = ssum_ref[0] * inv_n
    meanB = ssum_ref[1] * inv_n
    varA = ssq_ref[0] * inv_n - meanA * meanA
    varB = ssq_ref[1] * inv_n - meanB * meanB
    scaleA = gam_ref[0] * lax.rsqrt(varA + 1e-5)
    scaleB = gam_ref[1] * lax.rsqrt(varB + 1e-5)
    shiftA = bet_ref[0] - meanA * scaleA
    shiftB = bet_ref[1] - meanB * scaleB
    hA = jnp.maximum(hA_ref[...] * scaleA + shiftA, 0.0)
    hB = jnp.maximum(hB_ref[...] * scaleB + shiftB, 0.0)
    h = jnp.concatenate([hA, hB], axis=1)
    deg = (deg_ref[0, :, 0:1] + deg_ref[1, :, 0:1])
    a_ref[...] = jnp.dot(h, W1_ref[0], preferred_element_type=jnp.float32) \
        + b1_ref[0]
    rest_ref[...] = (jnp.dot(h, W3_ref[0], preferred_element_type=jnp.float32)
                     + b3_ref[0]
                     - deg * jnp.dot(h, W2_ref[0],
                                     preferred_element_type=jnp.float32))


def _combine_body(agg_ref, rest_ref, h_ref, ssum_ref, ssq_ref):
    i = pl.program_id(1)
    hp = agg_ref[...] + rest_ref[...]
    h_ref[...] = hp
    bs = jnp.sum(hp, axis=0, keepdims=True)
    bq = jnp.sum(hp * hp, axis=0, keepdims=True)

    @pl.when(i == 0)
    def _():
        ssum_ref[...] = bs[None]
        ssq_ref[...] = bq[None]

    @pl.when(i > 0)
    def _():
        ssum_ref[...] += bs[None]
        ssq_ref[...] += bq[None]


def _norm_body(h_ref, ssum_ref, ssq_ref, gam_ref, bet_ref, out_ref):
    inv_n = 1.0 / N
    mean = ssum_ref[0] * inv_n
    var = ssq_ref[0] * inv_n - mean * mean
    scale = gam_ref[0] * lax.rsqrt(var + 1e-5)
    shift = bet_ref[0] - mean * scale
    out_ref[...] = jnp.maximum(h_ref[...] * scale + shift, 0.0)


def _head_body(sum_ref, cnt_ref, W1_ref, b1_ref, W2_ref, b2_ref, out_ref):
    sums = jnp.concatenate([sum_ref[0], sum_ref[1]], axis=1)
    cnt = cnt_ref[0, :, 0:1]
    gx = sums / jnp.maximum(cnt, 1.0)
    p = jnp.maximum(jnp.dot(gx, W1_ref[...],
                            preferred_element_type=jnp.float32)
                    + b1_ref[...], 0.0)
    out_ref[...] = jnp.dot(p, W2_ref[...],
                           preferred_element_type=jnp.float32) + b2_ref[...]


def _row_spec(im):
    return pl.BlockSpec((RB, H), im)


_W_spec = pl.BlockSpec((1, D, H), lambda c, i: (c, 0, 0))
_b_spec = pl.BlockSpec((1, 1, H), lambda c, i: (c, 0, 0))
_deg_spec = pl.BlockSpec((NC, RB, 16), lambda c, i: (0, i, 0))
_stat_spec = pl.BlockSpec((1, 1, H), lambda c, i: (c, 0, 0))
_f32 = jnp.float32


def kernel(x, EdgeID, EdgeAttr, batch, emb_W, emb_b, lin1_W, lin1_b, lin2_W,
           lin3_W, lin3_b, bn_gamma, bn_beta, fc1_W, fc1_b, fc2_W, fc2_b):
    src = EdgeID[0].astype(jnp.int32)
    dst = EdgeID[1].astype(jnp.int32)
    batch = batch.astype(jnp.int32)

    z32 = jnp.zeros((RB, H), jnp.float32)
    z16 = jnp.zeros((RB, 16), jnp.float32)
    ones_rows = jnp.ones((C, 16), jnp.float32)

    pad = EP - E
    src_p = jnp.concatenate([src, jnp.zeros((pad,), jnp.int32)])
    dst_p = jnp.concatenate([dst, jnp.zeros((pad,), jnp.int32)])
    w_p = jnp.concatenate([EdgeAttr, jnp.zeros((pad,), jnp.float32)])
    ebuf2 = jnp.concatenate(
        [src_p.reshape(-1, C), dst_p.reshape(-1, C),
         jax.lax.bitcast_convert_type(w_p, jnp.int32).reshape(-1, C)], axis=1)

    deg2 = _deg_kernel(ebuf2, z16)

    def stack_w(W):
        return jnp.stack([W[:, :H], W[:, H:]], axis=0)

    def stack_b(b):
        return b.reshape(NC, 1, H)

    out_ab = [jax.ShapeDtypeStruct((NC * N, H), _f32)] * 2
    ab_specs = [_row_spec(lambda c, i: (c * NB + i, 0))] * 2

    a2, rest2 = pl.pallas_call(
        _mm_first_body,
        grid=(NC, NB),
        in_specs=[
            pl.BlockSpec((RB, 4), lambda c, i: (i, 0)),
            pl.BlockSpec((4, D), lambda c, i: (0, 0)),
            pl.BlockSpec((1, D), lambda c, i: (0, 0)),
            _W_spec, _b_spec, _W_spec, _W_spec, _b_spec, _deg_spec,
        ],
        out_specs=ab_specs,
        out_shape=out_ab,
    )(x, emb_W, emb_b.reshape(1, D), stack_w(lin1_W[0]), stack_b(lin1_b[0]),
      stack_w(lin2_W[0]), stack_w(lin3_W[0]), stack_b(lin3_b[0]), deg2)

    hpre2 = ssum = ssq = None
    for i in range(3):
        agg2 = _spmm_kernel(a2, ebuf2, z32)
        hpre2, ssum, ssq = pl.pallas_call(
            _combine_body,
            grid=(NC, NB),
            in_specs=ab_specs,
            out_specs=[ab_specs[0], _stat_spec, _stat_spec],
            out_shape=[jax.ShapeDtypeStruct((NC * N, H), _f32),
                       jax.ShapeDtypeStruct((NC, 1, H), _f32),
                       jax.ShapeDtypeStruct((NC, 1, H), _f32)],
        )(agg2, rest2)
        if i < 2:
            a2, rest2 = pl.pallas_call(
                _mm_body,
                grid=(NC, NB),
                in_specs=[
                    _row_spec(lambda c, i: (i, 0)),
                    _row_spec(lambda c, i: (NB + i, 0)),
                    pl.BlockSpec((NC, 1, H), lambda c, i: (0, 0, 0)),
                    pl.BlockSpec((NC, 1, H), lambda c, i: (0, 0, 0)),
                    pl.BlockSpec((NC, 1, H), lambda c, i: (0, 0, 0)),
                    pl.BlockSpec((NC, 1, H), lambda c, i: (0, 0, 0)),
                    _W_spec, _b_spec, _W_spec, _W_spec, _b_spec, _deg_spec,
                ],
                out_specs=ab_specs,
                out_shape=out_ab,
            )(hpre2, hpre2, ssum, ssq,
              bn_gamma[i].reshape(NC, 1, H), bn_beta[i].reshape(NC, 1, H),
              stack_w(lin1_W[i + 1]), stack_b(lin1_b[i + 1]),
              stack_w(lin2_W[i + 1]), stack_w(lin3_W[i + 1]),
              stack_b(lin3_b[i + 1]), deg2)

    hfin2 = pl.pallas_call(
        _norm_body,
        grid=(NC, NB),
        in_specs=[ab_specs[0], _stat_spec, _stat_spec, _stat_spec, _stat_spec],
        out_specs=ab_specs[0],
        out_shape=jax.ShapeDtypeStruct((NC * N, H), _f32),
    )(hpre2, ssum, ssq, bn_gamma[2].reshape(NC, 1, H),
      bn_beta[2].reshape(NC, 1, H))

    gsum, gcnt = _pool_kernel(hfin2, batch, z32, z16, ones_rows)

    out = pl.pallas_call(
        _head_body,
        in_specs=[
            pl.BlockSpec((NC, G, H), lambda: (0, 0, 0)),
            pl.BlockSpec((NC, G, 16), lambda: (0, 0, 0)),
            pl.BlockSpec((D, D), lambda: (0, 0)),
            pl.BlockSpec((1, D), lambda: (0, 0)),
            pl.BlockSpec((D, 3), lambda: (0, 0)),
            pl.BlockSpec((1, 3), lambda: (0, 0)),
        ],
        out_specs=pl.BlockSpec((G, 3), lambda: (0, 0)),
        out_shape=jax.ShapeDtypeStruct((G, 3), _f32),
    )(gsum, gcnt, fc1_W, fc1_b.reshape(1, D), fc2_W, fc2_b.reshape(1, 3))
    return out


# speedup vs baseline: 9.4538x; 1.3993x over previous
import functools

import jax
import jax.numpy as jnp
from jax import lax
from jax.experimental import pallas as pl
from jax.experimental.pallas import tpu as pltpu
from jax.experimental.pallas import tpu_sc as plsc

N = 50000
NP = 51200
E = 800000
G = 512
D = 64
H = 32
NC = 2
NS = 16
C = 128
SRB = 1600
PB = 512
NB = 25
PR = NP * H // 128
_PAD_ROW = (N * H // 128) - (NB - 1) * PB
_f32 = jnp.float32

_mesh = plsc.VectorSubcoreMesh(core_axis_name="c", subcore_axis_name="s",
                               num_cores=NC, num_subcores=NS)
_sc_params = pltpu.CompilerParams(use_tc_tiling_on_sc=False,
                                  needs_layout_passes=False)


@functools.partial(
    pl.kernel,
    out_type=jax.ShapeDtypeStruct((NC, NP, H), jnp.float32),
    mesh=_mesh,
    compiler_params=_sc_params,
    scratch_types=[
        pltpu.VMEM((3 * C,), jnp.int32),
        pltpu.VMEM((C,), jnp.int32),
        pltpu.VMEM((C, H), jnp.float32),
        pltpu.VMEM_SHARED((NP, H), jnp.float32),
        pltpu.SemaphoreType.DMA,
    ],
)
def _deg_kernel(ebuf_hbm, z32_hbm, out_hbm, ebuf, dstb, msgb, acc, sem):
    c = lax.axis_index("c")
    s = lax.axis_index("s")

    def zcopy(k, _):
        idx = s + k * NS

        @pl.when(idx < NP // SRB)
        def _():
            pltpu.sync_copy(z32_hbm, acc.at[pl.ds(idx * SRB, SRB)])
        return 0
    lax.fori_loop(0, pl.cdiv(NP // SRB, NS), zcopy, 0)
    plsc.subcore_barrier()

    kps = 6272 // NC // NS
    ch0 = c * (6272 // NC) + s * kps

    def chunk_body(k, _):
        pltpu.sync_copy(ebuf_hbm.at[ch0 + k], ebuf)

        def adj(j, _):
            dstb[pl.ds(j * 16, 16)] = ebuf[pl.ds(C + j * 16, 16)]
            return 0
        lax.fori_loop(0, C // 16, adj, 0)

        def scale(g, _):
            w16 = plsc.bitcast(ebuf[pl.ds(2 * C + g * 16, 16)], jnp.float32)
            for l in range(16):
                wv = jnp.full((16,), w16[l], jnp.float32)
                msgb[g * 16 + l, 0:16] = wv
                msgb[g * 16 + l, 16:32] = wv
            return 0
        lax.fori_loop(0, C // 16, scale, 0)
        pltpu.sync_copy(msgb, acc.at[dstb], add=True)
        return 0
    lax.fori_loop(0, kps, chunk_body, 0)
    plsc.subcore_barrier()

    def out_copy(k, _):
        idx = s + k * NS

        @pl.when(idx < NP // SRB)
        def _():
            pltpu.sync_copy(acc.at[pl.ds(idx * SRB, SRB)],
                            out_hbm.at[c, pl.ds(idx * SRB, SRB)])
        return 0
    lax.fori_loop(0, pl.cdiv(NP // SRB, NS), out_copy, 0)


EP = 6272 * C
_KPS = (EP // C) // NS


@functools.partial(
    pl.kernel,
    out_type=jax.ShapeDtypeStruct((NC * NP, H), jnp.float32),
    mesh=_mesh,
    compiler_params=_sc_params,
    scratch_types=[
        [pltpu.VMEM((3 * C,), jnp.int32)] * 2,
        [pltpu.VMEM((C,), jnp.int32)] * 2,
        [pltpu.VMEM((C,), jnp.int32)] * 2,
        [pltpu.VMEM((C, H), jnp.float32)] * 2,
        [pltpu.SemaphoreType.DMA] * 2,
        [pltpu.SemaphoreType.DMA] * 2,
        [pltpu.SemaphoreType.DMA] * 2,
        pltpu.VMEM_SHARED((NP, H), jnp.float32),
    ],
)
def _spmm_kernel(a2_hbm, ebuf_hbm, rest_hbm, out_hbm,
                 ebuf, srcadj, dstb, rows, sem_e, sem_g, sem_s, acc):
    c = lax.axis_index("c")
    s = lax.axis_index("s")
    row_off = c * NP

    def pre(k, _):
        idx = s + k * NS

        @pl.when(idx < NP // SRB)
        def _():
            pltpu.sync_copy(rest_hbm.at[pl.ds(row_off + idx * SRB, SRB)],
                            acc.at[pl.ds(idx * SRB, SRB)])
        return 0
    lax.fori_loop(0, pl.cdiv(NP // SRB, NS), pre, 0)
    plsc.subcore_barrier()

    ch0 = s * _KPS

    def unpack(b, ebv):
        def adj(j, _):
            srcadj[b][pl.ds(j * 16, 16)] = ebv[pl.ds(j * 16, 16)] + row_off
            dstb[b][pl.ds(j * 16, 16)] = ebv[pl.ds(C + j * 16, 16)]
            return 0
        lax.fori_loop(0, C // 16, adj, 0)

    def scale(b, ebv):
        def body(g, _):
            w16 = plsc.bitcast(ebv[pl.ds(2 * C + g * 16, 16)], jnp.float32)
            for l in range(16):
                e = g * 16 + l
                rows[b][e, 0:16] = rows[b][e, 0:16] * w16[l]
                rows[b][e, 16:32] = rows[b][e, 16:32] * w16[l]
            return 0
        lax.fori_loop(0, C // 16, body, 0)

    pltpu.sync_copy(ebuf_hbm.at[ch0], ebuf[0])
    unpack(0, ebuf[0])
    pltpu.async_copy(a2_hbm.at[srcadj[0]], rows[0], sem_g[0])
    pltpu.async_copy(ebuf_hbm.at[ch0 + 1], ebuf[1], sem_e[1])

    def visit(k, b):
        nxt = 1 - b

        @pl.when(k + 1 < _KPS)
        def _():
            pltpu.make_async_copy(ebuf_hbm.at[ch0], ebuf[nxt],
                                  sem_e[nxt]).wait()

            @pl.when(k >= 1)
            def _():
                pltpu.make_async_copy(rows[nxt], acc.at[dstb[nxt]],
                                      sem_s[nxt]).wait()
            unpack(nxt, ebuf[nxt])
            pltpu.async_copy(a2_hbm.at[srcadj[nxt]], rows[nxt], sem_g[nxt])
        pltpu.make_async_copy(a2_hbm.at[srcadj[b]], rows[b], sem_g[b]).wait()
        scale(b, ebuf[b])

        @pl.when(k + 2 < _KPS)
        def _():
            pltpu.async_copy(ebuf_hbm.at[ch0 + k + 2], ebuf[b], sem_e[b])
        pltpu.async_copy(rows[b], acc.at[dstb[b]], sem_s[b], add=True)

    def pair(kk, _):
        visit(kk * 2, 0)
        visit(kk * 2 + 1, 1)
        return 0
    lax.fori_loop(0, _KPS // 2, pair, 0)
    pltpu.make_async_copy(rows[0], acc.at[dstb[0]], sem_s[0]).wait()
    pltpu.make_async_copy(rows[1], acc.at[dstb[1]], sem_s[1]).wait()
    plsc.subcore_barrier()

    def out_copy(k, _):
        idx = s + k * NS

        @pl.when(idx < NP // SRB)
        def _():
            pltpu.sync_copy(acc.at[pl.ds(idx * SRB, SRB)],
                            out_hbm.at[pl.ds(row_off + idx * SRB, SRB)])
        return 0
    lax.fori_loop(0, pl.cdiv(NP // SRB, NS), out_copy, 0)


_NFULL = N // C
_REM = N - _NFULL * C


@functools.partial(
    pl.kernel,
    out_type=(jax.ShapeDtypeStruct((NC, G, H), jnp.float32),
              jax.ShapeDtypeStruct((NC, G, 16), jnp.float32)),
    mesh=_mesh,
    compiler_params=_sc_params,
    scratch_types=[
        pltpu.VMEM((C,), jnp.int32),
        pltpu.VMEM((C, H), jnp.float32),
        pltpu.VMEM((C, 16), jnp.float32),
        pltpu.VMEM((_REM,), jnp.int32),
        pltpu.VMEM((_REM, H), jnp.float32),
        pltpu.VMEM((_REM, 16), jnp.float32),
        pltpu.VMEM_SHARED((G, H), jnp.float32),
        pltpu.VMEM_SHARED((G, 16), jnp.float32),
        pltpu.SemaphoreType.DMA,
    ],
)
def _pool_kernel(h2_hbm, batch_hbm, z32_hbm, z16_hbm, ones_hbm,
                 sum_hbm, cnt_hbm,
                 bb, rows, ones, bb2, rows2, ones2, accS, accC, sem):
    c = lax.axis_index("c")
    s = lax.axis_index("s")

    @pl.when(s == 0)
    def _():
        pltpu.sync_copy(z32_hbm.at[pl.ds(0, G)], accS)
        pltpu.sync_copy(z16_hbm.at[pl.ds(0, G)], accC)

    pltpu.sync_copy(ones_hbm, ones)
    pltpu.sync_copy(ones_hbm.at[pl.ds(0, _REM)], ones2)
    plsc.subcore_barrier()

    row_off = c * NP

    def chunk_body(k, _):
        chunk = s + k * NS

        @pl.when(chunk < _NFULL)
        def _():
            base = chunk * C
            pltpu.sync_copy(batch_hbm.at[pl.ds(base, C)], bb)
            pltpu.sync_copy(h2_hbm.at[pl.ds(row_off + base, C)], rows)
            pltpu.sync_copy(rows, accS.at[bb], add=True)
            pltpu.sync_copy(ones, accC.at[bb], add=True)
        return 0
    lax.fori_loop(0, pl.cdiv(_NFULL, NS), chunk_body, 0)

    @pl.when(s == NS - 1)
    def _():
        base = _NFULL * C
        pltpu.sync_copy(batch_hbm.at[pl.ds(base, _REM)], bb2)
        pltpu.sync_copy(h2_hbm.at[pl.ds(row_off + base, _REM)], rows2)
        pltpu.sync_copy(rows2, accS.at[bb2], add=True)
        pltpu.sync_copy(ones2, accC.at[bb2], add=True)
    plsc.subcore_barrier()

    @pl.when(s == 0)
    def _():
        pltpu.sync_copy(accS, sum_hbm.at[c])
        pltpu.sync_copy(accC, cnt_hbm.at[c])


def _pad_mask(rest):
    row = (pl.program_id(1) * PB
           + lax.broadcasted_iota(jnp.int32, (PB, 128), 0))
    return jnp.where(row >= N * H // 128, 0.0, rest)


def _mm_first_body(xp_ref, R_ref, bias_ref, dgA_ref, dgB_ref,
                   a_ref, rest_ref):
    m = jnp.dot(xp_ref[...], R_ref[0], precision=jax.lax.Precision.HIGHEST,
                preferred_element_type=_f32) + bias_ref[0]
    degm = dgA_ref[...] + dgB_ref[...]
    a_ref[...] = m[:, 0:128]
    rest_ref[...] = _pad_mask(m[:, 128:256] - degm * m[:, 256:384])


def _mm_body(hA_ref, hB_ref, ssum_ref, ssq_ref, gam_ref, bet_ref,
             RA_ref, RB_ref, bias_ref, dgA_ref, dgB_ref,
             a_ref, rest_ref):
    inv_n = 1.0 / N
    meanA = ssum_ref[0] * inv_n
    meanB = ssum_ref[1] * inv_n
    varA = ssq_ref[0] * inv_n - meanA * meanA
    varB = ssq_ref[1] * inv_n - meanB * meanB
    scaleA = gam_ref[0] * lax.rsqrt(varA + 1e-5)
    scaleB = gam_ref[1] * lax.rsqrt(varB + 1e-5)
    shiftA = bet_ref[0] - meanA * scaleA
    shiftB = bet_ref[1] - meanB * scaleB
    hA = jnp.maximum(hA_ref[...] * scaleA + shiftA, 0.0)
    hB = jnp.maximum(hB_ref[...] * scaleB + shiftB, 0.0)
    hp = jax.lax.Precision.HIGHEST
    m = (jnp.dot(hA, RA_ref[0], precision=hp, preferred_element_type=_f32)
         + jnp.dot(hB, RB_ref[0], precision=hp, preferred_element_type=_f32)
         + bias_ref[0])
    degm = dgA_ref[...] + dgB_ref[...]
    a_ref[...] = m[:, 0:128]
    rest_ref[...] = _pad_mask(m[:, 128:256] - degm * m[:, 256:384])


def _stats_body(h_ref, fold_ref, ssum_ref, ssq_ref):
    i = pl.program_id(1)
    hp = h_ref[...]
    bs = jnp.sum(hp, axis=0, keepdims=True)
    bq = jnp.sum(hp * hp, axis=0, keepdims=True)

    @pl.when(i == 0)
    def _():
        ssum_ref[...] = bs[None]
        ssq_ref[...] = bq[None]

    @pl.when(i > 0)
    def _():
        ssum_ref[...] += bs[None]
        ssq_ref[...] += bq[None]

    @pl.when(i == NB - 1)
    def _():
        hi = jax.lax.Precision.HIGHEST
        ssum_ref[...] = jnp.dot(ssum_ref[0], fold_ref[...],
                                precision=hi,
                                preferred_element_type=_f32)[None]
        ssq_ref[...] = jnp.dot(ssq_ref[0], fold_ref[...],
                               precision=hi,
                               preferred_element_type=_f32)[None]


def _norm_body(h_ref, ssum_ref, ssq_ref, gam_ref, bet_ref, out_ref):
    inv_n = 1.0 / N
    mean = ssum_ref[0] * inv_n
    var = ssq_ref[0] * inv_n - mean * mean
    scale = gam_ref[0] * lax.rsqrt(var + 1e-5)
    shift = bet_ref[0] - mean * scale
    out_ref[...] = jnp.maximum(h_ref[...] * scale + shift, 0.0)


def _head_body(sum_ref, cnt_ref, W1_ref, b1_ref, W2_ref, b2_ref, out_ref):
    sums = jnp.concatenate([sum_ref[0], sum_ref[1]], axis=1)
    cnt = cnt_ref[0, :, 0:1]
    gx = sums / jnp.maximum(cnt, 1.0)
    p = jnp.maximum(jnp.dot(gx, W1_ref[...],
                            preferred_element_type=_f32)
                    + b1_ref[...], 0.0)
    out_ref[...] = jnp.dot(p, W2_ref[...],
                           preferred_element_type=_f32) + b2_ref[...]


_pk_spec_c = pl.BlockSpec((PB, 128), lambda c, i: (c * NB + i, 0))
_pk_spec_A = pl.BlockSpec((PB, 128), lambda c, i: (i, 0))
_pk_spec_B = pl.BlockSpec((PB, 128), lambda c, i: (NB + i, 0))
_stat_spec = pl.BlockSpec((NC, 1, 128), lambda c, i: (0, 0, 0))
_stat_spec_c = pl.BlockSpec((1, 1, 128), lambda c, i: (c, 0, 0))
_R_spec = pl.BlockSpec((1, 128, 384), lambda c, i: (c, 0, 0))
_bias_spec = pl.BlockSpec((1, 1, 384), lambda c, i: (c, 0, 0))
_pk_out = jax.ShapeDtypeStruct((NC * PR, 128), _f32)
_stat_out = jax.ShapeDtypeStruct((NC, 1, 128), _f32)


def _kron4(W):
    return jnp.kron(jnp.eye(4, dtype=_f32), W)


def _rhs(W1, W3, W2, c, r0):
    sl = slice(c * H, (c + 1) * H)
    rs = slice(r0, r0 + H)
    return jnp.concatenate(
        [_kron4(W1[rs, sl]), _kron4(W3[rs, sl]), _kron4(W2[rs, sl])], axis=1)


def _tile4(v):
    return jnp.tile(v.reshape(1, -1), (1, 4))


def kernel(x, EdgeID, EdgeAttr, batch, emb_W, emb_b, lin1_W, lin1_b, lin2_W,
           lin3_W, lin3_b, bn_gamma, bn_beta, fc1_W, fc1_b, fc2_W, fc2_b):
    src = EdgeID[0].astype(jnp.int32)
    dst = EdgeID[1].astype(jnp.int32)
    batch = batch.astype(jnp.int32)

    z32 = jnp.zeros((SRB, H), _f32)
    z16 = jnp.zeros((SRB, 16), _f32)
    ones_rows = jnp.ones((C, 16), _f32)

    pad = EP - E
    src_p = jnp.concatenate([src, jnp.zeros((pad,), jnp.int32)])
    dst_p = jnp.concatenate([dst, jnp.zeros((pad,), jnp.int32)])
    w_p = jnp.concatenate([EdgeAttr, jnp.zeros((pad,), _f32)])
    ebuf2 = jnp.concatenate(
        [src_p.reshape(-1, C), dst_p.reshape(-1, C),
         jax.lax.bitcast_convert_type(w_p, jnp.int32).reshape(-1, C)], axis=1)

    fold_mat = jnp.kron(jnp.ones((4, 4), _f32), jnp.eye(H, dtype=_f32))

    deg2 = _deg_kernel(ebuf2, z32)
    degp = deg2.reshape(NC * PR, 128)

    hi = jax.lax.Precision.HIGHEST
    effW1 = jnp.dot(emb_W, lin1_W[0], precision=hi)
    effW2 = jnp.dot(emb_W, lin2_W[0], precision=hi)
    effW3 = jnp.dot(emb_W, lin3_W[0], precision=hi)
    effb1 = jnp.dot(emb_b, lin1_W[0], precision=hi) + lin1_b[0]
    effb2 = jnp.dot(emb_b, lin2_W[0], precision=hi)
    effb3 = jnp.dot(emb_b, lin3_W[0], precision=hi) + lin3_b[0]

    def rhs1(c):
        sl = slice(c * H, (c + 1) * H)
        return jnp.concatenate(
            [jnp.kron(jnp.eye(4, dtype=_f32), effW1[:, sl]),
             jnp.kron(jnp.eye(4, dtype=_f32), effW3[:, sl]),
             jnp.kron(jnp.eye(4, dtype=_f32), effW2[:, sl])], axis=1)

    R1 = jnp.stack([rhs1(0), rhs1(1)])
    bias1 = jnp.stack([
        jnp.concatenate([_tile4(effb1[c * H:(c + 1) * H]),
                         _tile4(effb3[c * H:(c + 1) * H]),
                         _tile4(effb2[c * H:(c + 1) * H])], axis=1)
        for c in range(NC)])

    x_pad = jnp.concatenate([x, jnp.zeros((NP - N, 4), _f32)])
    xp = x_pad.reshape(-1, 16)

    a2p, restp = pl.pallas_call(
        _mm_first_body,
        grid=(NC, NB),
        in_specs=[
            pl.BlockSpec((PB, 16), lambda c, i: (i, 0)),
            pl.BlockSpec((1, 16, 384), lambda c, i: (c, 0, 0)),
            _bias_spec, _pk_spec_A, _pk_spec_B,
        ],
        out_specs=[_pk_spec_c, _pk_spec_c],
        out_shape=[_pk_out, _pk_out],
    )(xp, R1, bias1, degp, degp)

    hprep = ssum = ssq = None
    for i in range(3):
        hpre2 = _spmm_kernel(a2p.reshape(NC * NP, H), ebuf2,
                             restp.reshape(NC * NP, H))
        hprep = hpre2.reshape(NC * PR, 128)
        ssum, ssq = pl.pallas_call(
            _stats_body,
            grid=(NC, NB),
            in_specs=[_pk_spec_c,
                      pl.BlockSpec((128, 128), lambda c, i: (0, 0))],
            out_specs=[_stat_spec_c, _stat_spec_c],
            out_shape=[_stat_out, _stat_out],
        )(hprep, fold_mat)
        if i < 2:
            RA = jnp.stack([_rhs(lin1_W[i + 1], lin3_W[i + 1],
                                 lin2_W[i + 1], c, 0) for c in range(NC)])
            RBm = jnp.stack([_rhs(lin1_W[i + 1], lin3_W[i + 1],
                                  lin2_W[i + 1], c, H) for c in range(NC)])
            biasL = jnp.stack([
                jnp.concatenate([_tile4(lin1_b[i + 1, c * H:(c + 1) * H]),
                                 _tile4(lin3_b[i + 1, c * H:(c + 1) * H]),
                                 jnp.zeros((1, 128), _f32)], axis=1)
                for c in range(NC)])
            gamT = jnp.stack([_tile4(bn_gamma[i, c * H:(c + 1) * H])
                              for c in range(NC)])
            betT = jnp.stack([_tile4(bn_beta[i, c * H:(c + 1) * H])
                              for c in range(NC)])
            a2p, restp = pl.pallas_call(
                _mm_body,
                grid=(NC, NB),
                in_specs=[
                    _pk_spec_A, _pk_spec_B,
                    _stat_spec, _stat_spec,
                    pl.BlockSpec((NC, 1, 128), lambda c, i: (0, 0, 0)),
                    pl.BlockSpec((NC, 1, 128), lambda c, i: (0, 0, 0)),
                    _R_spec, _R_spec, _bias_spec,
                    _pk_spec_A, _pk_spec_B,
                ],
                out_specs=[_pk_spec_c, _pk_spec_c],
                out_shape=[_pk_out, _pk_out],
            )(hprep, hprep, ssum, ssq, gamT, betT, RA, RBm, biasL,
              degp, degp)

    gamT = jnp.stack([_tile4(bn_gamma[2, c * H:(c + 1) * H])
                      for c in range(NC)])
    betT = jnp.stack([_tile4(bn_beta[2, c * H:(c + 1) * H])
                      for c in range(NC)])
    hfinp = pl.pallas_call(
        _norm_body,
        grid=(NC, NB),
        in_specs=[_pk_spec_c, _stat_spec_c, _stat_spec_c,
                  _stat_spec_c, _stat_spec_c],
        out_specs=_pk_spec_c,
        out_shape=_pk_out,
    )(hprep, ssum, ssq, gamT, betT)

    gsum, gcnt = _pool_kernel(hfinp.reshape(NC * NP, H), batch,
                              z32, z16, ones_rows)

    out = pl.pallas_call(
        _head_body,
        in_specs=[
            pl.BlockSpec((NC, G, H), lambda: (0, 0, 0)),
            pl.BlockSpec((NC, G, 16), lambda: (0, 0, 0)),
            pl.BlockSpec((D, D), lambda: (0, 0)),
            pl.BlockSpec((1, D), lambda: (0, 0)),
            pl.BlockSpec((D, 3), lambda: (0, 0)),
            pl.BlockSpec((1, 3), lambda: (0, 0)),
        ],
        out_specs=pl.BlockSpec((G, 3), lambda: (0, 0)),
        out_shape=jax.ShapeDtypeStruct((G, 3), _f32),
    )(gsum, gcnt, fc1_W, fc1_b.reshape(1, D), fc2_W, fc2_b.reshape(1, 3))
    return out
